# Initial kernel scaffold; baseline (speedup 1.0000x reference)
import jax
import jax.numpy as jnp
from jax.experimental import pallas as pl


def kernel(x, edge_index, edge_attr, batch, W_ne, b_ne, g_ne, be_ne, W_rel, W_root, b_conv, g_ln, be_ln, W_ee, b_ee, g_ee, be_ee, W_gate, b_gate, W_a1, b_a1, W_a2, b_a2, W_t1, b_t1, g_t, be_t, W_m1, b_m1, g_m, be_m, W_m2, b_m2):
    raise NotImplementedError("write your pallas kernel here")


# speedup vs baseline: 1.8653x; 1.8653x over previous
import functools

import jax
import jax.numpy as jnp
from jax import lax
from jax.experimental import pallas as pl
from jax.experimental.pallas import tpu as pltpu
from jax.experimental.pallas import tpu_sc as plsc

F32 = jnp.float32
I32 = jnp.int32

N = 10000
E = 160000
H = 128
DE = 16
R = 4
NL = 3
G = 16

K = 128
NC, NS = 2, 16
NW = NC * NS
EP = 163840
CHT = EP // (NW * K)
NP = 10240
N4 = N * R
N4P = 40960

_EPS = 1e-5


def _ln(x, g, b):
    mu = jnp.mean(x, axis=-1, keepdims=True)
    var = jnp.mean((x - mu) ** 2, axis=-1, keepdims=True)
    return (x - mu) / jnp.sqrt(var + _EPS) * g + b


BN = 2000
BE = 2048


def _enc_body(x_ref, wne, bne, gne, bene, wcat, h_ref, hc_ref):
    h = jnp.maximum(x_ref[...] @ wne[...] + bne[...], 0.0)
    h = _ln(h, gne[...], bene[...])
    h_ref[...] = h
    hc_ref[...] = h @ wcat[...]


def _eprep_body(ea_ref, src_ref, dst_ref, gi_ref, gi2_ref):
    pid = pl.program_id(0)
    ea4 = ea_ref[:, 0:4]
    best = jnp.max(ea4, axis=1, keepdims=True)
    io4 = lax.broadcasted_iota(I32, (BE, 4), 1)
    cand = jnp.where(ea4 == best, io4, 99)
    t = jnp.min(cand, axis=1, keepdims=True)
    lin = pid * BE + lax.broadcasted_iota(I32, (BE, 1), 0)
    valid = lin < E
    gi_ref[...] = jnp.where(valid, src_ref[...] * R + t, 0)
    gi2_ref[...] = jnp.where(valid, dst_ref[...] * R + t, N4)


def _invcnt_body(cnt_ref, inv_ref):
    pid = pl.program_id(0)
    c = cnt_ref[...]
    lin = (pid * 8 + lax.broadcasted_iota(I32, (8, 128), 0)) * 128 + \
        lax.broadcasted_iota(I32, (8, 128), 1)
    inv_ref[...] = jnp.where(lin < N4, 1.0 / jnp.maximum(c, 1.0), 0.0)


def _layer_body(h_ref, p_ref, wroot, bconv, gln, beln, wcat, hn_ref, hc_ref):
    o = h_ref[...] @ wroot[...] + bconv[...] + p_ref[...]
    hn = h_ref[...] + jnp.maximum(_ln(o, gln[...], beln[...]), 0.0)
    hn_ref[...] = hn
    hc_ref[...] = hn @ wcat[...]


def _layer3_body(h_ref, p_ref, wroot, bconv, gln, beln, wgate, bgate,
                 batch_ref, hn_ref, gate_ref, gmax_ref):
    pid = pl.program_id(0)
    o = h_ref[...] @ wroot[...] + bconv[...] + p_ref[...]
    hn = h_ref[...] + jnp.maximum(_ln(o, gln[...], beln[...]), 0.0)
    hn_ref[...] = hn
    gate = hn @ wgate[...] + bgate[...]
    gate_ref[...] = gate

    @pl.when(pid == 0)
    def _():
        gmax_ref[...] = jnp.full((8, G), -1e30, F32)

    onehot = batch_ref[...] == lax.broadcasted_iota(I32, (BN, G), 1)
    m = jnp.where(onehot, gate, -1e30)
    pmax = jnp.max(m, axis=0)
    gmax_ref[...] = jnp.maximum(gmax_ref[...], jnp.broadcast_to(pmax, (8, G)))


def _pool_body(h_ref, gate_ref, batch_ref, gmax_ref, wa1, wt1,
               ge_ref, ga4_ref, gt4_ref, t_acc, s_acc):
    pid = pl.program_id(0)

    @pl.when(pid == 0)
    def _():
        t_acc[...] = jnp.zeros((G, H), F32)
        s_acc[...] = jnp.zeros((G, H), F32)

    onehot = (batch_ref[...] ==
              lax.broadcasted_iota(I32, (BN, G), 1)).astype(F32)
    gmn = onehot @ gmax_ref[0:1, :].reshape(G, 1)
    eg = jnp.exp(gate_ref[...] - gmn)
    egh = eg * h_ref[...]
    dn = (((0,), (0,)), ((), ()))
    t_acc[...] += lax.dot_general(onehot, egh, dn)
    s_acc[...] += lax.dot_general(onehot, jnp.broadcast_to(eg, (BN, H)), dn)
    ge = t_acc[...] / (s_acc[...] + 1e-16)
    ge_ref[...] = ge
    ga4_ref[...] = ge @ wa1[3 * H:4 * H, :]
    gt4_ref[...] = ge @ wt1[3 * H:4 * H, :]


def _head_body(ns_ref, nd_ref, ea_ref, bsrc_ref, ga4_ref, gt4_ref,
               wee, bee, gee, beee, wa1, ba1, wa2, ba2,
               wt1, bt1, gt, bet, wm1, bm1, gm, bem, wm2, bm2, out_ref):
    ee = _ln(jnp.maximum(ea_ref[...] @ wee[...] + bee[...], 0.0),
             gee[...], beee[...])
    onehot = (bsrc_ref[...] ==
              lax.broadcasted_iota(I32, (BE, G), 1)).astype(F32)
    ns = ns_ref[...]
    nd = nd_ref[...]
    pre_a = (ns @ wa1[0:H, :] + nd @ wa1[H:2 * H, :] + ee @ wa1[2 * H:3 * H, :]
             + onehot @ ga4_ref[...] + ba1[...])
    scores = jax.nn.sigmoid(jnp.maximum(pre_a, 0.0) @ wa2[...] + ba2[...])
    pre_t = (ns @ wt1[0:H, :] + nd @ wt1[H:2 * H, :] + ee @ wt1[2 * H:3 * H, :]
             + onehot @ gt4_ref[...] + bt1[...])
    att = _ln(jnp.maximum(pre_t, 0.0), gt[...], bet[...]) * scores
    m = _ln(jnp.maximum(att @ wm1[...] + bm1[...], 0.0), gm[...], bem[...])
    out_ref[...] = m @ wm2[...] + bm2[...]


def _full(shape):
    nd = len(shape)
    return pl.BlockSpec(shape, lambda i, *_: (0,) * nd)


def _tc(body, grid, in_specs, out_specs, out_shape, scratch_shapes=()):
    return pl.pallas_call(
        body, grid=grid, in_specs=in_specs, out_specs=out_specs,
        out_shape=out_shape, scratch_shapes=scratch_shapes)


def _mesh():
    return plsc.VectorSubcoreMesh(core_axis_name="c", subcore_axis_name="s",
                                  num_cores=NC, num_subcores=NS)

NZC = N4P // NS
NZA = NP // NS


def _sc_counts(gi2, zer):
    @functools.partial(
        pl.kernel,
        out_type=jax.ShapeDtypeStruct((NW, N4P // 128, 128), F32),
        mesh=_mesh(),
        compiler_params=pltpu.CompilerParams(needs_layout_passes=False),
        scratch_types=[pltpu.VMEM((K,), I32),
                       pltpu.VMEM((N4P // 128, 128), F32)],
    )
    def k(gi2_hbm, zer_hbm, out_hbm, giv, cntv):
        c = lax.axis_index("c")
        s = lax.axis_index("s")
        wid = c * NS + s
        pltpu.sync_copy(zer_hbm, cntv)
        one = jnp.full((16,), 1.0, F32)

        def chunk(j, _):
            off = (wid * CHT + j) * K
            pltpu.sync_copy(gi2_hbm.at[pl.ds(off, K)], giv)

            def sub(t, _):
                idx = giv[pl.ds(t * 16, 16)]
                plsc.addupdate_scatter(cntv, [idx >> 7, idx & 127], one)
                return 0
            lax.fori_loop(0, K // 16, sub, 0)
            return 0
        lax.fori_loop(0, CHT, chunk, 0)
        pltpu.sync_copy(cntv, out_hbm.at[wid])

    return k(gi2, zer)


def _sc_w_bsrc(invc, batchp, gi2, srcp):
    @functools.partial(
        pl.kernel,
        out_type=[jax.ShapeDtypeStruct((EP,), F32),
                  jax.ShapeDtypeStruct((EP,), I32)],
        mesh=_mesh(),
        compiler_params=pltpu.CompilerParams(needs_layout_passes=False),
        scratch_types=[pltpu.VMEM((N4P // 128, 128), F32),
                       pltpu.VMEM((NP // 128, 128), I32),
                       pltpu.VMEM((K,), I32), pltpu.VMEM((K,), I32),
                       pltpu.VMEM((K,), F32), pltpu.VMEM((K,), I32)],
    )
    def k(invc_hbm, batch_hbm, gi2_hbm, src_hbm, w_hbm, bsrc_hbm,
          invcv, batchv, g2v, srcv, wv, bv):
        c = lax.axis_index("c")
        s = lax.axis_index("s")
        pltpu.sync_copy(invc_hbm, invcv)
        pltpu.sync_copy(batch_hbm, batchv)

        def chunk(j, _):
            off = ((c * NS + s) * CHT + j) * K
            pltpu.sync_copy(gi2_hbm.at[pl.ds(off, K)], g2v)
            pltpu.sync_copy(src_hbm.at[pl.ds(off, K)], srcv)

            def sub(t, _):
                idx = g2v[pl.ds(t * 16, 16)]
                wv[pl.ds(t * 16, 16)] = plsc.load_gather(
                    invcv, [idx >> 7, idx & 127])
                idx2 = srcv[pl.ds(t * 16, 16)]
                bv[pl.ds(t * 16, 16)] = plsc.load_gather(
                    batchv, [idx2 >> 7, idx2 & 127])
                return 0
            lax.fori_loop(0, K // 16, sub, 0)
            pltpu.sync_copy(wv, w_hbm.at[pl.ds(off, K)])
            pltpu.sync_copy(bv, bsrc_hbm.at[pl.ds(off, K)])
            return 0
        lax.fori_loop(0, CHT, chunk, 0)

    return k(invc, batchp, gi2, srcp)


NSEG = 2560
NACC = 2688
NZP = NACC // NS
CHM = EP // (NS * K)
PH = 2


def _sc_msg(hcat4, gi, w, dstloc, zer):
    @functools.partial(
        pl.kernel,
        out_type=jax.ShapeDtypeStruct((PH * NC, NACC, H), F32),
        mesh=_mesh(),
        compiler_params=pltpu.CompilerParams(needs_layout_passes=False),
        scratch_types=[pltpu.VMEM((K,), I32), pltpu.VMEM((K,), F32),
                       pltpu.VMEM((K, H), F32), pltpu.VMEM((CHM, K), I32),
                       pltpu.VMEM((NZP, H), F32),
                       pltpu.VMEM_SHARED((NACC, H), F32),
                       pltpu.SemaphoreType.DMA],
    )
    def k(hcat_hbm, gi_hbm, w_hbm, dl_hbm, zer_hbm, out_hbm,
          giv, wv, rows, dstall, zb, acc, sem):
        c = lax.axis_index("c")
        s = lax.axis_index("s")
        pltpu.sync_copy(zer_hbm, zb)
        for p in range(PH):
            q = p * NC + c
            pltpu.sync_copy(zb, acc.at[pl.ds(s * NZP, NZP)])
            pltpu.sync_copy(dl_hbm.at[q, pl.ds(s * CHM, CHM)], dstall)
            plsc.subcore_barrier()

            def chunk(j, _):
                off = (s * CHM + j) * K
                pltpu.sync_copy(gi_hbm.at[pl.ds(off, K)], giv)
                pltpu.async_copy(hcat_hbm.at[giv], rows, sem).wait()
                pltpu.sync_copy(w_hbm.at[pl.ds(off, K)], wv)

                def srow(jj, _):
                    sp = plsc.load_gather(wv, [jnp.full((16,), 1, I32) * jj])
                    for cc in range(8):
                        sl = pl.ds(cc * 16, 16)
                        rows[jj, sl] = rows[jj, sl] * sp
                    return 0
                lax.fori_loop(0, K, srow, 0)
                pltpu.sync_copy(rows, acc.at[dstall.at[j]], add=True)
                return 0
            lax.fori_loop(0, CHM, chunk, 0)
            plsc.subcore_barrier()
            pltpu.sync_copy(acc.at[pl.ds(s * NZP, NZP)], zb)
            pltpu.sync_copy(zb, out_hbm.at[q, pl.ds(s * NZP, NZP)])
            pltpu.sync_copy(zer_hbm, zb)

    return k(hcat4, gi, w, dstloc, zer)


def _sc_gather_nodes(h3, srcp, dstp):
    @functools.partial(
        pl.kernel,
        out_type=[jax.ShapeDtypeStruct((EP, H), F32),
                  jax.ShapeDtypeStruct((EP, H), F32)],
        mesh=_mesh(),
        compiler_params=pltpu.CompilerParams(needs_layout_passes=False),
        scratch_types=[pltpu.VMEM((K,), I32), pltpu.VMEM((K,), I32),
                       pltpu.VMEM((K, H), F32), pltpu.VMEM((K, H), F32),
                       pltpu.SemaphoreType.DMA],
    )
    def k(h_hbm, src_hbm, dst_hbm, ns_hbm, nd_hbm, siv, div, rs, rd, sem):
        c = lax.axis_index("c")
        s = lax.axis_index("s")

        def chunk(j, _):
            off = ((c * NS + s) * CHT + j) * K
            pltpu.sync_copy(src_hbm.at[pl.ds(off, K)], siv)
            pltpu.async_copy(h_hbm.at[siv], rs, sem).wait()
            pltpu.sync_copy(rs, ns_hbm.at[pl.ds(off, K)])
            pltpu.sync_copy(dst_hbm.at[pl.ds(off, K)], div)
            pltpu.async_copy(h_hbm.at[div], rd, sem).wait()
            pltpu.sync_copy(rd, nd_hbm.at[pl.ds(off, K)])
            return 0
        lax.fori_loop(0, CHT, chunk, 0)

    return k(h3, srcp, dstp)


def kernel(x, edge_index, edge_attr, batch, W_ne, b_ne, g_ne, be_ne, W_rel,
           W_root, b_conv, g_ln, be_ln, W_ee, b_ee, g_ee, be_ee, W_gate,
           b_gate, W_a1, b_a1, W_a2, b_a2, W_t1, b_t1, g_t, be_t, W_m1, b_m1,
           g_m, be_m, W_m2, b_m2):
    src = edge_index[0].astype(I32)
    dst = edge_index[1].astype(I32)
    padE = EP - E
    srcp = jnp.concatenate([src, jnp.zeros((padE,), I32)])
    dstp = jnp.concatenate([dst, jnp.zeros((padE,), I32)])
    eap = jnp.concatenate([edge_attr, jnp.zeros((padE, DE), F32)])
    batchp = jnp.concatenate([batch.astype(I32),
                              jnp.zeros((NP - N,), I32)])

    wcat = [jnp.transpose(W_rel[i], (1, 0, 2)).reshape(H, R * H)
            for i in range(NL)]
    r1 = lambda v: v.reshape(1, -1)
    zer_cnt = jnp.zeros((N4P // 128, 128), F32)
    zer_acc = jnp.zeros((NZP, H), F32)

    grid_n = (N // BN,)
    bspec_h = pl.BlockSpec((BN, H), lambda i: (i, 0))
    bspec_hc = pl.BlockSpec((BN, R * H), lambda i: (i, 0))
    h0, hcat1 = _tc(
        _enc_body, grid_n,
        [pl.BlockSpec((BN, H), lambda i: (i, 0)), _full((H, H)),
         _full((1, H)), _full((1, H)), _full((1, H)), _full((H, R * H))],
        [bspec_h, bspec_hc],
        [jax.ShapeDtypeStruct((N, H), F32),
         jax.ShapeDtypeStruct((N, R * H), F32)],
    )(x, W_ne, r1(b_ne), r1(g_ne), r1(be_ne), wcat[0])

    grid_e = (EP // BE,)
    bspec_e1 = pl.BlockSpec((BE, 1), lambda i: (i, 0))
    gi, gi2 = _tc(
        _eprep_body, grid_e,
        [pl.BlockSpec((BE, DE), lambda i: (i, 0)), bspec_e1, bspec_e1],
        [bspec_e1, bspec_e1],
        [jax.ShapeDtypeStruct((EP, 1), I32),
         jax.ShapeDtypeStruct((EP, 1), I32)],
    )(eap, srcp.reshape(EP, 1), dstp.reshape(EP, 1))
    gi = gi.reshape(EP)
    gi2 = gi2.reshape(EP)

    cnt01 = _sc_counts(gi2, zer_cnt)
    cnt_sum = jnp.sum(cnt01, axis=0)
    invc = _tc(
        _invcnt_body, (N4P // 1024,),
        [pl.BlockSpec((8, 128), lambda i: (i, 0))],
        pl.BlockSpec((8, 128), lambda i: (i, 0)),
        jax.ShapeDtypeStruct((N4P // 128, 128), F32),
    )(cnt_sum)
    w, bsrc = _sc_w_bsrc(invc, batchp.reshape(NP // 128, 128), gi2, srcp)

    seg = dstp // NSEG
    dstloc = jnp.stack([jnp.where(seg == q, dstp - q * NSEG, NSEG)
                        for q in range(PH * NC)]).reshape(PH * NC, EP // K, K)
    h = h0
    hcat = hcat1
    for i in range(NL):
        part = _sc_msg(hcat.reshape(N4, H), gi, w, dstloc, zer_acc)
        psum = jnp.concatenate([part[q, :NSEG] for q in range(PH * NC)])[:N]
        bspec_p = pl.BlockSpec((BN, H), lambda i: (i, 0))
        if i < NL - 1:
            h, hcat = _tc(
                _layer_body, grid_n,
                [bspec_h, bspec_p, _full((H, H)), _full((1, H)),
                 _full((1, H)), _full((1, H)), _full((H, R * H))],
                [bspec_h, bspec_hc],
                [jax.ShapeDtypeStruct((N, H), F32),
                 jax.ShapeDtypeStruct((N, R * H), F32)],
            )(h, psum, W_root[i], r1(b_conv[i]), r1(g_ln[i]), r1(be_ln[i]),
              wcat[i + 1])
        else:
            h, gate, gmax8 = _tc(
                _layer3_body, grid_n,
                [bspec_h, bspec_p, _full((H, H)), _full((1, H)),
                 _full((1, H)), _full((1, H)), _full((H, 1)), _full((1, 1)),
                 pl.BlockSpec((BN, 1), lambda i: (i, 0))],
                [bspec_h, pl.BlockSpec((BN, 1), lambda i: (i, 0)),
                 _full((8, G))],
                [jax.ShapeDtypeStruct((N, H), F32),
                 jax.ShapeDtypeStruct((N, 1), F32),
                 jax.ShapeDtypeStruct((8, G), F32)],
            )(h, psum, W_root[i], r1(b_conv[i]), r1(g_ln[i]), r1(be_ln[i]),
              W_gate, b_gate.reshape(1, 1), batch.astype(I32).reshape(N, 1))

    ge, ga4, gt4 = _tc(
        _pool_body, grid_n,
        [bspec_h, pl.BlockSpec((BN, 1), lambda i: (i, 0)),
         pl.BlockSpec((BN, 1), lambda i: (i, 0)), _full((8, G)),
         _full((4 * H, H)), _full((4 * H, H))],
        [_full((G, H)), _full((G, H)), _full((G, H))],
        [jax.ShapeDtypeStruct((G, H), F32),
         jax.ShapeDtypeStruct((G, H), F32),
         jax.ShapeDtypeStruct((G, H), F32)],
        scratch_shapes=[pltpu.VMEM((G, H), F32), pltpu.VMEM((G, H), F32)],
    )(h, gate, batch.astype(I32).reshape(N, 1), gmax8, W_a1, W_t1)

    ns, nd = _sc_gather_nodes(h, srcp, dstp)

    pred = _tc(
        _head_body, grid_e,
        [pl.BlockSpec((BE, H), lambda i: (i, 0)),
         pl.BlockSpec((BE, H), lambda i: (i, 0)),
         pl.BlockSpec((BE, DE), lambda i: (i, 0)), bspec_e1,
         _full((G, H)), _full((G, H)),
         _full((DE, H)), _full((1, H)), _full((1, H)), _full((1, H)),
         _full((4 * H, H)), _full((1, H)), _full((H, 1)), _full((1, 1)),
         _full((4 * H, H)), _full((1, H)), _full((1, H)), _full((1, H)),
         _full((H, H // 2)), _full((1, H // 2)), _full((1, H // 2)),
         _full((1, H // 2)), _full((H // 2, 1)), _full((1, 1))],
        bspec_e1,
        jax.ShapeDtypeStruct((EP, 1), F32),
    )(ns, nd, eap, bsrc.reshape(EP, 1), ga4, gt4,
      W_ee, r1(b_ee), r1(g_ee), r1(be_ee),
      W_a1, r1(b_a1), W_a2, b_a2.reshape(1, 1),
      W_t1, r1(b_t1), r1(g_t), r1(be_t),
      W_m1, r1(b_m1), r1(g_m), r1(be_m), W_m2, b_m2.reshape(1, 1))

    return pred.reshape(EP)[:E]
